# speedup vs baseline: 3.9953x; 3.9953x over previous
import functools

import jax
import jax.numpy as jnp
from jax.experimental import pallas as pl


def _match_block_kernel(onehot_ref, gtp_ref, logits_ref, pboxes_ref,
                        min_ref, idx_ref, *, qb: int, gpad: int):
    qi = pl.program_id(1)

    l = logits_ref[0]
    m = jnp.max(l, axis=-1, keepdims=True)
    e = jnp.exp(l - m)
    s = jnp.sum(e, axis=-1, keepdims=True)
    eg = jnp.dot(e, onehot_ref[0], preferred_element_type=jnp.float32)
    prob = eg * (1.0 / s)

    pb = pboxes_ref[0]
    pcx, pcy, pw, ph = (pb[:, 0:1], pb[:, 1:2], pb[:, 2:3], pb[:, 3:4])
    px1 = pcx - 0.5 * pw
    py1 = pcy - 0.5 * ph
    px2 = pcx + 0.5 * pw
    py2 = pcy + 0.5 * ph
    parea = (px2 - px1) * (py2 - py1)

    g = gtp_ref[0]
    gcx, gcy, gw, gh = (g[0:1, :], g[1:2, :], g[2:3, :], g[3:4, :])
    gx1, gy1, gx2, gy2 = (g[4:5, :], g[5:6, :], g[6:7, :], g[7:8, :])
    garea = g[8:9, :]

    bbox = (jnp.abs(pcx - gcx) + jnp.abs(pcy - gcy)
            + jnp.abs(pw - gw) + jnp.abs(ph - gh))

    lt_x = jnp.maximum(px1, gx1)
    lt_y = jnp.maximum(py1, gy1)
    rb_x = jnp.minimum(px2, gx2)
    rb_y = jnp.minimum(py2, gy2)
    iw = jnp.maximum(rb_x - lt_x, 0.0)
    ih = jnp.maximum(rb_y - lt_y, 0.0)
    inter = iw * ih
    union = parea + garea - inter
    iou = inter / (union + 1e-8)
    ex1 = jnp.minimum(px1, gx1)
    ey1 = jnp.minimum(py1, gy1)
    ex2 = jnp.maximum(px2, gx2)
    ey2 = jnp.maximum(py2, gy2)
    ew = jnp.maximum(ex2 - ex1, 0.0)
    eh = jnp.maximum(ey2 - ey1, 0.0)
    earea = ew * eh
    giou = iou - (earea - union) / (earea + 1e-8)

    cost = 5.0 * bbox + (-prob) + 2.0 * (-giou)

    blk_min = jnp.min(cost, axis=0, keepdims=True)
    qidx = (qi * qb
            + jax.lax.broadcasted_iota(jnp.int32, (qb, gpad), 0))
    big = jnp.int32(2**30)
    blk_idx = jnp.min(jnp.where(cost == blk_min, qidx, big),
                      axis=0, keepdims=True)

    @pl.when(qi == 0)
    def _init():
        min_ref[0] = jnp.full_like(min_ref[0], jnp.inf)
        idx_ref[0] = jnp.zeros_like(idx_ref[0])

    acc_min = min_ref[0]
    better = blk_min < acc_min
    min_ref[0] = jnp.where(better, blk_min, acc_min)
    idx_ref[0] = jnp.where(better, blk_idx, idx_ref[0])


@functools.partial(jax.jit, static_argnames=("qb", "interpret"))
def _match_tc(pred_logits, pred_boxes, gt_boxes, gt_labels, qb=1000,
              interpret=False):
    B, Q, C = pred_logits.shape
    G = gt_labels.shape[1]
    nq = Q // qb

    onehot = (gt_labels[:, None, :] ==
              jnp.arange(C, dtype=gt_labels.dtype)[None, :, None]
              ).astype(jnp.float32)
    gcx, gcy, gw, gh = (gt_boxes[..., 0], gt_boxes[..., 1],
                        gt_boxes[..., 2], gt_boxes[..., 3])
    gx1 = gcx - 0.5 * gw
    gy1 = gcy - 0.5 * gh
    gx2 = gcx + 0.5 * gw
    gy2 = gcy + 0.5 * gh
    garea = (gx2 - gx1) * (gy2 - gy1)
    zeros = jnp.zeros_like(gcx)
    gtp = jnp.stack([gcx, gcy, gw, gh, gx1, gy1, gx2, gy2, garea,
                     zeros, zeros, zeros, zeros, zeros, zeros, zeros],
                    axis=1)

    kern = functools.partial(_match_block_kernel, qb=qb, gpad=G)
    min_c, idx = pl.pallas_call(
        kern,
        grid=(B, nq),
        in_specs=[
            pl.BlockSpec((1, C, G), lambda b, qi: (b, 0, 0)),
            pl.BlockSpec((1, 16, G), lambda b, qi: (b, 0, 0)),
            pl.BlockSpec((1, qb, C), lambda b, qi: (b, qi, 0)),
            pl.BlockSpec((1, qb, 4), lambda b, qi: (b, qi, 0)),
        ],
        out_specs=[
            pl.BlockSpec((1, 1, G), lambda b, qi: (b, 0, 0)),
            pl.BlockSpec((1, 1, G), lambda b, qi: (b, 0, 0)),
        ],
        out_shape=[
            jax.ShapeDtypeStruct((B, 1, G), jnp.float32),
            jax.ShapeDtypeStruct((B, 1, G), jnp.int32),
        ],
        interpret=interpret,
    )(onehot, gtp, pred_logits, pred_boxes)
    return min_c[:, 0, :], idx[:, 0, :]


def kernel(pred_logits, pred_boxes, gt_boxes, gt_labels):
    return _match_tc(pred_logits, pred_boxes, gt_boxes, gt_labels)


# speedup vs baseline: 5.2210x; 1.3068x over previous
import functools

import jax
import jax.numpy as jnp
from jax.experimental import pallas as pl

_TILE = 8
_BIG = 2**30


def _match_block_kernel(onehot_ref, gtrep_ref, logits_ref, pboxes_ref,
                        min_ref, idx_ref, *, qb: int, g: int):
    qi = pl.program_id(1)

    l = logits_ref[0]
    m = jnp.max(l, axis=-1, keepdims=True)
    e = jnp.exp(l - m)
    s = jnp.sum(e, axis=-1, keepdims=True)

    eg = jnp.dot(e, onehot_ref[0], preferred_element_type=jnp.float32,
                 precision=jax.lax.Precision.HIGHEST)

    pb = pboxes_ref[0]

    gt = gtrep_ref[0]
    bgcx, bgcy, bgw, bgh = (gt[0:8], gt[8:16], gt[16:24], gt[24:32])
    bgx1, bgy1, bgx2, bgy2 = (gt[32:40], gt[40:48], gt[48:56], gt[56:64])
    bgarea = gt[64:72]

    ntiles = qb // _TILE
    accs = []
    for k in range(4):
        accs.append([jnp.full((_TILE, g), jnp.inf, jnp.float32),
                     jnp.zeros((_TILE, g), jnp.int32)])
    base_iota = jax.lax.broadcasted_iota(jnp.int32, (_TILE, g), 0)

    for i in range(ntiles):
        sl = slice(i * _TILE, (i + 1) * _TILE)
        pbt = pb[sl, :]
        bcx = jnp.broadcast_to(pbt[:, 0:1], (_TILE, g))
        bcy = jnp.broadcast_to(pbt[:, 1:2], (_TILE, g))
        bw = jnp.broadcast_to(pbt[:, 2:3], (_TILE, g))
        bh = jnp.broadcast_to(pbt[:, 3:4], (_TILE, g))
        bs = jnp.broadcast_to(s[sl, :], (_TILE, g))
        prob = eg[sl, :] / bs

        hw = 0.5 * bw
        hh = 0.5 * bh
        px1 = bcx - hw
        py1 = bcy - hh
        px2 = bcx + hw
        py2 = bcy + hh
        parea = (px2 - px1) * (py2 - py1)

        bbox = (jnp.abs(bcx - bgcx) + jnp.abs(bcy - bgcy)
                + jnp.abs(bw - bgw) + jnp.abs(bh - bgh))

        iw = jnp.maximum(jnp.minimum(px2, bgx2) - jnp.maximum(px1, bgx1), 0.0)
        ih = jnp.maximum(jnp.minimum(py2, bgy2) - jnp.maximum(py1, bgy1), 0.0)
        inter = iw * ih
        union = parea + bgarea - inter
        iou = inter / (union + 1e-8)
        ew = jnp.maximum(px2, bgx2) - jnp.minimum(px1, bgx1)
        eh = jnp.maximum(py2, bgy2) - jnp.minimum(py1, bgy1)
        ew = jnp.maximum(ew, 0.0)
        eh = jnp.maximum(eh, 0.0)
        earea = ew * eh
        giou = iou - (earea - union) / (earea + 1e-8)

        cost = 5.0 * bbox + (-prob) + 2.0 * (-giou)

        qidx = base_iota + (qi * qb + i * _TILE)
        am, ai = accs[i % 4]
        lt = cost < am
        accs[i % 4] = [jnp.where(lt, cost, am), jnp.where(lt, qidx, ai)]

    blk_min = jnp.minimum(jnp.minimum(accs[0][0], accs[1][0]),
                          jnp.minimum(accs[2][0], accs[3][0]))
    blk_min = jnp.min(blk_min, axis=0, keepdims=True)
    cand = jnp.full((1, g), _BIG, jnp.int32)
    for am, ai in accs:
        masked = jnp.where(am == blk_min, ai, _BIG)
        cand = jnp.minimum(cand, jnp.min(masked, axis=0, keepdims=True))

    @pl.when(qi == 0)
    def _init():
        min_ref[0] = jnp.full_like(min_ref[0], jnp.inf)
        idx_ref[0] = jnp.zeros_like(idx_ref[0])

    acc_min = min_ref[0]
    better = blk_min < acc_min
    min_ref[0] = jnp.where(better, blk_min, acc_min)
    idx_ref[0] = jnp.where(better, cand, idx_ref[0])


@functools.partial(jax.jit, static_argnames=("qb", "interpret"))
def _match_tc(pred_logits, pred_boxes, gt_boxes, gt_labels, qb=1000,
              interpret=False):
    B, Q, C = pred_logits.shape
    G = gt_labels.shape[1]
    nq = Q // qb

    onehot = (gt_labels[:, None, :] ==
              jnp.arange(C, dtype=gt_labels.dtype)[None, :, None]
              ).astype(jnp.float32)
    gcx, gcy, gw, gh = (gt_boxes[..., 0], gt_boxes[..., 1],
                        gt_boxes[..., 2], gt_boxes[..., 3])
    gx1 = gcx - 0.5 * gw
    gy1 = gcy - 0.5 * gh
    gx2 = gcx + 0.5 * gw
    gy2 = gcy + 0.5 * gh
    garea = (gx2 - gx1) * (gy2 - gy1)
    gtrep = jnp.stack([gcx, gcy, gw, gh, gx1, gy1, gx2, gy2, garea],
                      axis=1)
    gtrep = jnp.repeat(gtrep, _TILE, axis=1)

    kern = functools.partial(_match_block_kernel, qb=qb, g=G)
    min_c, idx = pl.pallas_call(
        kern,
        grid=(B, nq),
        in_specs=[
            pl.BlockSpec((1, C, G), lambda b, qi: (b, 0, 0)),
            pl.BlockSpec((1, 72, G), lambda b, qi: (b, 0, 0)),
            pl.BlockSpec((1, qb, C), lambda b, qi: (b, qi, 0)),
            pl.BlockSpec((1, qb, 4), lambda b, qi: (b, qi, 0)),
        ],
        out_specs=[
            pl.BlockSpec((1, 1, G), lambda b, qi: (b, 0, 0)),
            pl.BlockSpec((1, 1, G), lambda b, qi: (b, 0, 0)),
        ],
        out_shape=[
            jax.ShapeDtypeStruct((B, 1, G), jnp.float32),
            jax.ShapeDtypeStruct((B, 1, G), jnp.int32),
        ],
        interpret=interpret,
    )(onehot, gtrep, pred_logits, pred_boxes)
    return min_c[:, 0, :], idx[:, 0, :]


def kernel(pred_logits, pred_boxes, gt_boxes, gt_labels):
    return _match_tc(pred_logits, pred_boxes, gt_boxes, gt_labels)


# speedup vs baseline: 5.4409x; 1.0421x over previous
import functools

import jax
import jax.numpy as jnp
from jax.experimental import pallas as pl

_TILE = 8
_BIG = 2**30


def _match_block_kernel(onehot_ref, gtrep_ref, logits_ref, pboxes_ref,
                        min_ref, idx_ref, *, qb: int, g: int):
    qi = pl.program_id(1)

    l = logits_ref[0]
    m = jnp.max(l, axis=-1, keepdims=True)
    e = jnp.exp(l - m)
    s = jnp.sum(e, axis=-1, keepdims=True)

    eg = jnp.dot(e, onehot_ref[0], preferred_element_type=jnp.float32,
                 precision=jax.lax.Precision.HIGHEST)

    pb = pboxes_ref[0]

    gt = gtrep_ref[0]
    bgcx, bgcy, bgw, bgh = (gt[0:8], gt[8:16], gt[16:24], gt[24:32])
    bgx1, bgy1, bgx2, bgy2 = (gt[32:40], gt[40:48], gt[48:56], gt[56:64])
    bgarea = gt[64:72]

    ntiles = qb // _TILE
    accs = []
    for k in range(4):
        accs.append([jnp.full((_TILE, g), jnp.inf, jnp.float32),
                     jnp.zeros((_TILE, g), jnp.int32)])
    base_iota = jax.lax.broadcasted_iota(jnp.int32, (_TILE, g), 0)

    for i in range(ntiles):
        sl = slice(i * _TILE, (i + 1) * _TILE)
        pbt = pb[sl, :]
        bcx = jnp.broadcast_to(pbt[:, 0:1], (_TILE, g))
        bcy = jnp.broadcast_to(pbt[:, 1:2], (_TILE, g))
        bw = jnp.broadcast_to(pbt[:, 2:3], (_TILE, g))
        bh = jnp.broadcast_to(pbt[:, 3:4], (_TILE, g))
        bs = jnp.broadcast_to(s[sl, :], (_TILE, g))
        prob = eg[sl, :] / bs

        hw = 0.5 * bw
        hh = 0.5 * bh
        px1 = bcx - hw
        py1 = bcy - hh
        px2 = bcx + hw
        py2 = bcy + hh
        parea = (px2 - px1) * (py2 - py1)

        bbox = (jnp.abs(bcx - bgcx) + jnp.abs(bcy - bgcy)
                + jnp.abs(bw - bgw) + jnp.abs(bh - bgh))

        iw = jnp.maximum(jnp.minimum(px2, bgx2) - jnp.maximum(px1, bgx1), 0.0)
        ih = jnp.maximum(jnp.minimum(py2, bgy2) - jnp.maximum(py1, bgy1), 0.0)
        inter = iw * ih
        union = parea + bgarea - inter
        iou = inter / (union + 1e-8)
        ew = jnp.maximum(px2, bgx2) - jnp.minimum(px1, bgx1)
        eh = jnp.maximum(py2, bgy2) - jnp.minimum(py1, bgy1)
        ew = jnp.maximum(ew, 0.0)
        eh = jnp.maximum(eh, 0.0)
        earea = ew * eh
        giou = iou - (earea - union) / (earea + 1e-8)

        cost = 5.0 * bbox + (-prob) + 2.0 * (-giou)

        qidx = base_iota + (qi * qb + i * _TILE)
        am, ai = accs[i % 4]
        lt = cost < am
        accs[i % 4] = [jnp.where(lt, cost, am), jnp.where(lt, qidx, ai)]

    blk_min = jnp.minimum(jnp.minimum(accs[0][0], accs[1][0]),
                          jnp.minimum(accs[2][0], accs[3][0]))
    blk_min = jnp.min(blk_min, axis=0, keepdims=True)
    cand = jnp.full((1, g), _BIG, jnp.int32)
    for am, ai in accs:
        masked = jnp.where(am == blk_min, ai, _BIG)
        cand = jnp.minimum(cand, jnp.min(masked, axis=0, keepdims=True))

    @pl.when(qi == 0)
    def _init():
        min_ref[0] = jnp.full_like(min_ref[0], jnp.inf)
        idx_ref[0] = jnp.zeros_like(idx_ref[0])

    acc_min = min_ref[0]
    better = blk_min < acc_min
    min_ref[0] = jnp.where(better, blk_min, acc_min)
    idx_ref[0] = jnp.where(better, cand, idx_ref[0])


@functools.partial(jax.jit, static_argnames=("qb", "interpret"))
def _match_tc(pred_logits, pred_boxes, gt_boxes, gt_labels, qb=2000,
              interpret=False):
    B, Q, C = pred_logits.shape
    G = gt_labels.shape[1]
    nq = Q // qb

    onehot = (gt_labels[:, None, :] ==
              jnp.arange(C, dtype=gt_labels.dtype)[None, :, None]
              ).astype(jnp.float32)
    gcx, gcy, gw, gh = (gt_boxes[..., 0], gt_boxes[..., 1],
                        gt_boxes[..., 2], gt_boxes[..., 3])
    gx1 = gcx - 0.5 * gw
    gy1 = gcy - 0.5 * gh
    gx2 = gcx + 0.5 * gw
    gy2 = gcy + 0.5 * gh
    garea = (gx2 - gx1) * (gy2 - gy1)
    gtrep = jnp.stack([gcx, gcy, gw, gh, gx1, gy1, gx2, gy2, garea],
                      axis=1)
    gtrep = jnp.repeat(gtrep, _TILE, axis=1)

    kern = functools.partial(_match_block_kernel, qb=qb, g=G)
    min_c, idx = pl.pallas_call(
        kern,
        grid=(B, nq),
        in_specs=[
            pl.BlockSpec((1, C, G), lambda b, qi: (b, 0, 0)),
            pl.BlockSpec((1, 72, G), lambda b, qi: (b, 0, 0)),
            pl.BlockSpec((1, qb, C), lambda b, qi: (b, qi, 0)),
            pl.BlockSpec((1, qb, 4), lambda b, qi: (b, qi, 0)),
        ],
        out_specs=[
            pl.BlockSpec((1, 1, G), lambda b, qi: (b, 0, 0)),
            pl.BlockSpec((1, 1, G), lambda b, qi: (b, 0, 0)),
        ],
        out_shape=[
            jax.ShapeDtypeStruct((B, 1, G), jnp.float32),
            jax.ShapeDtypeStruct((B, 1, G), jnp.int32),
        ],
        interpret=interpret,
    )(onehot, gtrep, pred_logits, pred_boxes)
    return min_c[:, 0, :], idx[:, 0, :]


def kernel(pred_logits, pred_boxes, gt_boxes, gt_labels):
    return _match_tc(pred_logits, pred_boxes, gt_boxes, gt_labels)
